# baseline (speedup 1.0000x reference)
import functools

import jax
import jax.numpy as jnp
from jax import lax
from jax.experimental import pallas as pl
from jax.experimental.pallas import tpu as pltpu
from jax.experimental.pallas import tpu_sc as plsc

_NC = 2
_NS = 16
_NW = _NC * _NS
_LANES = 16
_NBUF = 8
_CHUNK_B = 1
_VB = 2048


def _sc_pool(idx_hbm, table_hbm, h_hbm, scratch,
             *, T, D, b_per_w, chunk_b, n_chunks, chunk_pad):
    nvec = D // _LANES
    scale = jnp.float32(1.0 / T)
    idx_refs = scratch[:n_chunks]
    rows_bufs = scratch[n_chunks:n_chunks + _NBUF]
    hbuf = scratch[n_chunks + _NBUF]
    sems = scratch[n_chunks + _NBUF + 1:n_chunks + _NBUF + 1 + _NBUF]
    idx_sem = scratch[n_chunks + _NBUF + 1 + _NBUF]

    wid = lax.axis_index("s") * _NC + lax.axis_index("c")
    idx_cps = [pltpu.async_copy(idx_hbm.at[wid, ch], idx_refs[ch], idx_sem)
               for ch in range(n_chunks)]
    for cp in idx_cps:
        cp.wait()

    def start(ch):
        return pltpu.async_copy(
            table_hbm.at[idx_refs[ch]], rows_bufs[ch % _NBUF],
            sems[ch % _NBUF])

    cps = [None] * _NBUF
    for ch in range(min(_NBUF - 1, n_chunks)):
        cps[ch % _NBUF] = start(ch)
    for ch in range(n_chunks):
        nxt = ch + _NBUF - 1
        if nxt < n_chunks:
            cps[nxt % _NBUF] = start(nxt)
        cps[ch % _NBUF].wait()
        rows = rows_bufs[ch % _NBUF]
        for bl in range(chunk_b):
            base = bl * T

            def t_body(t, acc, rows=rows, base=base):
                r0 = base + 2 * t
                return tuple(acc[c]
                             + rows[r0, pl.ds(c * _LANES, _LANES)]
                             + rows[r0 + 1, pl.ds(c * _LANES, _LANES)]
                             for c in range(nvec))

            acc = lax.fori_loop(
                0, T // 2, t_body,
                tuple(jnp.zeros((_LANES,), jnp.float32) for _ in range(nvec)))
            row = ch * chunk_b + bl
            for c in range(nvec):
                last = rows[base + T - 1, pl.ds(c * _LANES, _LANES)] \
                    if T % 2 else 0.0
                hbuf[row, pl.ds(c * _LANES, _LANES)] = (acc[c] + last) * scale
    pltpu.sync_copy(hbuf, h_hbm.at[pl.ds(wid * b_per_w, b_per_w)])


def _pooled_embedding(idx_pad, embed_weight, *, B, T, D, b_per_w, chunk_b,
                      n_chunks, chunk_pad):
    mesh = plsc.VectorSubcoreMesh(core_axis_name="c", subcore_axis_name="s",
                                  num_cores=_NC, num_subcores=_NS)

    def body(idx_hbm, table_hbm, h_hbm, *scratch):
        _sc_pool(idx_hbm, table_hbm, h_hbm, scratch, T=T, D=D,
                 b_per_w=b_per_w, chunk_b=chunk_b, n_chunks=n_chunks,
                 chunk_pad=chunk_pad)

    scratch_types = (
        [pltpu.VMEM((chunk_pad,), jnp.int32)] * n_chunks
        + [pltpu.VMEM((chunk_pad, D), jnp.float32)] * _NBUF
        + [pltpu.VMEM((b_per_w, D), jnp.float32)]
        + [pltpu.SemaphoreType.DMA] * (_NBUF + 1)
    )
    return pl.kernel(
        body,
        out_type=jax.ShapeDtypeStruct((B, D), jnp.float32),
        mesh=mesh,
        scratch_types=scratch_types,
    )(idx_pad, embed_weight)


def _mm_body(w_ref, h_ref, b_ref, out_ref):
    acc = lax.dot_general(w_ref[...], h_ref[...],
                          (((1,), (1,)), ((), ())),
                          preferred_element_type=jnp.float32)
    i = pl.program_id(0)
    vb, nvb = b_ref.shape
    onehot = (lax.broadcasted_iota(jnp.int32, (vb, nvb), 1) == i)
    bcol = jnp.sum(jnp.where(onehot, b_ref[...], 0.0), axis=1, keepdims=True)
    out_ref[...] = acc + bcol


def _project_t(h, w, bias_cols, *, B, D, V, vb, nvb):
    return pl.pallas_call(
        _mm_body,
        grid=(nvb,),
        in_specs=[
            pl.BlockSpec((vb, D), lambda i: (i, 0)),
            pl.BlockSpec((B, D), lambda i: (0, 0)),
            pl.BlockSpec((vb, nvb), lambda i: (0, 0)),
        ],
        out_specs=pl.BlockSpec((vb, B), lambda i: (i, 0)),
        out_shape=jax.ShapeDtypeStruct((V, B), jnp.float32),
        compiler_params=pltpu.CompilerParams(
            dimension_semantics=("arbitrary",)),
    )(w, h, bias_cols)


def kernel(indices, embed_weight, proj_weight, proj_bias):
    B, T = indices.shape
    V, D = embed_weight.shape
    b_per_w = B // _NW
    chunk_b = _CHUNK_B
    chunk = chunk_b * T
    chunk_pad = (chunk + 7) // 8 * 8
    n_chunks = b_per_w // chunk_b

    idx = indices.astype(jnp.int32).reshape(_NW, n_chunks, chunk)
    idx_pad = jnp.pad(idx, ((0, 0), (0, 0), (0, chunk_pad - chunk)))

    h = _pooled_embedding(idx_pad, embed_weight, B=B, T=T, D=D,
                          b_per_w=b_per_w, chunk_b=chunk_b,
                          n_chunks=n_chunks, chunk_pad=chunk_pad)

    vb = _VB
    nvb = pl.cdiv(V, vb)
    bias_cols = jnp.pad(proj_bias, (0, nvb * vb - V)).reshape(nvb, vb).T
    logits_t = _project_t(h, proj_weight, bias_cols, B=B, D=D, V=V,
                          vb=vb, nvb=nvb)
    return logits_t.T


# speedup vs baseline: 1.5492x; 1.5492x over previous
import functools

import jax
import jax.numpy as jnp
from jax import lax
from jax.experimental import pallas as pl
from jax.experimental.pallas import tpu as pltpu
from jax.experimental.pallas import tpu_sc as plsc

_NC = 2
_NS = 16
_NW = _NC * _NS
_LANES = 16
_NBUF = 4
_CHUNK_B = 2
_VB = 4096


def _sc_pool(idx_hbm, table_hbm, h_hbm, scratch,
             *, T, D, b_per_w, chunk_b, n_chunks, chunk_pad):
    nvec = D // _LANES
    scale = jnp.float32(1.0 / T)
    idx_refs = scratch[:n_chunks]
    rows_bufs = scratch[n_chunks:n_chunks + _NBUF]
    hbuf = scratch[n_chunks + _NBUF]
    sems = scratch[n_chunks + _NBUF + 1:n_chunks + _NBUF + 1 + _NBUF]
    idx_sem = scratch[n_chunks + _NBUF + 1 + _NBUF]

    wid = lax.axis_index("s") * _NC + lax.axis_index("c")
    idx_cps = [pltpu.async_copy(idx_hbm.at[wid, ch], idx_refs[ch], idx_sem)
               for ch in range(n_chunks)]
    for cp in idx_cps:
        cp.wait()

    def start(ch):
        return pltpu.async_copy(
            table_hbm.at[idx_refs[ch]], rows_bufs[ch % _NBUF],
            sems[ch % _NBUF])

    cps = [None] * _NBUF
    for ch in range(min(_NBUF - 1, n_chunks)):
        cps[ch % _NBUF] = start(ch)
    for ch in range(n_chunks):
        nxt = ch + _NBUF - 1
        if nxt < n_chunks:
            cps[nxt % _NBUF] = start(nxt)
        cps[ch % _NBUF].wait()
        rows = rows_bufs[ch % _NBUF]
        for bl in range(chunk_b):
            base = bl * T

            def t_body(t, acc, rows=rows, base=base):
                r0 = base + 2 * t
                return tuple(acc[c]
                             + rows[r0, pl.ds(c * _LANES, _LANES)]
                             + rows[r0 + 1, pl.ds(c * _LANES, _LANES)]
                             for c in range(nvec))

            acc = lax.fori_loop(
                0, T // 2, t_body,
                tuple(jnp.zeros((_LANES,), jnp.float32) for _ in range(nvec)))
            row = ch * chunk_b + bl
            for c in range(nvec):
                last = rows[base + T - 1, pl.ds(c * _LANES, _LANES)] \
                    if T % 2 else 0.0
                hbuf[row, pl.ds(c * _LANES, _LANES)] = (acc[c] + last) * scale
    pltpu.sync_copy(hbuf, h_hbm.at[pl.ds(wid * b_per_w, b_per_w)])


def _pooled_embedding(idx_pad, embed_weight, *, B, T, D, b_per_w, chunk_b,
                      n_chunks, chunk_pad):
    mesh = plsc.VectorSubcoreMesh(core_axis_name="c", subcore_axis_name="s",
                                  num_cores=_NC, num_subcores=_NS)

    def body(idx_hbm, table_hbm, h_hbm, *scratch):
        _sc_pool(idx_hbm, table_hbm, h_hbm, scratch, T=T, D=D,
                 b_per_w=b_per_w, chunk_b=chunk_b, n_chunks=n_chunks,
                 chunk_pad=chunk_pad)

    scratch_types = (
        [pltpu.VMEM((chunk_pad,), jnp.int32)] * n_chunks
        + [pltpu.VMEM((chunk_pad, D), jnp.float32)] * _NBUF
        + [pltpu.VMEM((b_per_w, D), jnp.float32)]
        + [pltpu.SemaphoreType.DMA] * (_NBUF + 1)
    )
    return pl.kernel(
        body,
        out_type=jax.ShapeDtypeStruct((B, D), jnp.float32),
        mesh=mesh,
        scratch_types=scratch_types,
    )(idx_pad, embed_weight)


def _mm_body(w_ref, h_ref, b_ref, out_ref):
    acc = lax.dot_general(w_ref[...], h_ref[...],
                          (((1,), (1,)), ((), ())),
                          preferred_element_type=jnp.float32)
    i = pl.program_id(0)
    vb, nvb = b_ref.shape
    onehot = (lax.broadcasted_iota(jnp.int32, (vb, nvb), 1) == i)
    bcol = jnp.sum(jnp.where(onehot, b_ref[...], 0.0), axis=1, keepdims=True)
    out_ref[...] = acc + bcol


def _project_t(h, w, bias_cols, *, B, D, V, vb, nvb):
    return pl.pallas_call(
        _mm_body,
        grid=(nvb,),
        in_specs=[
            pl.BlockSpec((vb, D), lambda i: (i, 0)),
            pl.BlockSpec((B, D), lambda i: (0, 0)),
            pl.BlockSpec((vb, nvb), lambda i: (0, 0)),
        ],
        out_specs=pl.BlockSpec((vb, B), lambda i: (i, 0)),
        out_shape=jax.ShapeDtypeStruct((V, B), jnp.float32),
        compiler_params=pltpu.CompilerParams(
            dimension_semantics=("arbitrary",)),
    )(w, h, bias_cols)


def kernel(indices, embed_weight, proj_weight, proj_bias):
    B, T = indices.shape
    V, D = embed_weight.shape
    b_per_w = B // _NW
    chunk_b = _CHUNK_B
    chunk = chunk_b * T
    chunk_pad = (chunk + 7) // 8 * 8
    n_chunks = b_per_w // chunk_b

    idx = indices.astype(jnp.int32).reshape(_NW, n_chunks, chunk)
    idx_pad = jnp.pad(idx, ((0, 0), (0, 0), (0, chunk_pad - chunk)))

    h = _pooled_embedding(idx_pad, embed_weight, B=B, T=T, D=D,
                          b_per_w=b_per_w, chunk_b=chunk_b,
                          n_chunks=n_chunks, chunk_pad=chunk_pad)

    vb = _VB
    nvb = pl.cdiv(V, vb)
    bias_cols = jnp.pad(proj_bias, (0, nvb * vb - V)).reshape(nvb, vb).T
    logits_t = _project_t(h, proj_weight, bias_cols, B=B, D=D, V=V,
                          vb=vb, nvb=nvb)
    return logits_t.T


# speedup vs baseline: 2.1535x; 1.3901x over previous
import functools

import jax
import jax.numpy as jnp
from jax import lax
from jax.experimental import pallas as pl
from jax.experimental.pallas import tpu as pltpu
from jax.experimental.pallas import tpu_sc as plsc

_NC = 2
_NS = 16
_NW = _NC * _NS
_LANES = 16
_NBUF = 2
_CHUNK_B = 4
_VB = 4096


def _sc_pool(idx_hbm, table_hbm, h_hbm, scratch,
             *, T, D, b_per_w, chunk_b, n_chunks, chunk_pad):
    nvec = D // _LANES
    scale = jnp.float32(1.0 / T)
    idx_refs = scratch[:n_chunks]
    rows_bufs = scratch[n_chunks:n_chunks + _NBUF]
    hbuf = scratch[n_chunks + _NBUF]
    sems = scratch[n_chunks + _NBUF + 1:n_chunks + _NBUF + 1 + _NBUF]
    idx_sem = scratch[n_chunks + _NBUF + 1 + _NBUF]

    wid = lax.axis_index("s") * _NC + lax.axis_index("c")
    idx_cps = [pltpu.async_copy(idx_hbm.at[wid, ch], idx_refs[ch], idx_sem)
               for ch in range(n_chunks)]
    for cp in idx_cps:
        cp.wait()

    def start(ch):
        return pltpu.async_copy(
            table_hbm.at[idx_refs[ch]], rows_bufs[ch % _NBUF],
            sems[ch % _NBUF])

    cps = [None] * _NBUF
    for ch in range(min(_NBUF - 1, n_chunks)):
        cps[ch % _NBUF] = start(ch)
    for ch in range(n_chunks):
        nxt = ch + _NBUF - 1
        if nxt < n_chunks:
            cps[nxt % _NBUF] = start(nxt)
        cps[ch % _NBUF].wait()
        rows = rows_bufs[ch % _NBUF]
        for bl in range(chunk_b):
            base = bl * T

            def t_body(t, acc, rows=rows, base=base):
                r0 = base + 2 * t
                return tuple(acc[c]
                             + rows[r0, pl.ds(c * _LANES, _LANES)]
                             + rows[r0 + 1, pl.ds(c * _LANES, _LANES)]
                             for c in range(nvec))

            acc = lax.fori_loop(
                0, T // 2, t_body,
                tuple(jnp.zeros((_LANES,), jnp.float32) for _ in range(nvec)))
            row = ch * chunk_b + bl
            for c in range(nvec):
                last = rows[base + T - 1, pl.ds(c * _LANES, _LANES)] \
                    if T % 2 else 0.0
                hbuf[row, pl.ds(c * _LANES, _LANES)] = (acc[c] + last) * scale
    pltpu.sync_copy(hbuf, h_hbm.at[pl.ds(wid * b_per_w, b_per_w)])


def _pooled_embedding(idx_pad, embed_weight, *, B, T, D, b_per_w, chunk_b,
                      n_chunks, chunk_pad):
    mesh = plsc.VectorSubcoreMesh(core_axis_name="c", subcore_axis_name="s",
                                  num_cores=_NC, num_subcores=_NS)

    def body(idx_hbm, table_hbm, h_hbm, *scratch):
        _sc_pool(idx_hbm, table_hbm, h_hbm, scratch, T=T, D=D,
                 b_per_w=b_per_w, chunk_b=chunk_b, n_chunks=n_chunks,
                 chunk_pad=chunk_pad)

    scratch_types = (
        [pltpu.VMEM((chunk_pad,), jnp.int32)] * n_chunks
        + [pltpu.VMEM((chunk_pad, D), jnp.float32)] * _NBUF
        + [pltpu.VMEM((b_per_w, D), jnp.float32)]
        + [pltpu.SemaphoreType.DMA] * (_NBUF + 1)
    )
    return pl.kernel(
        body,
        out_type=jax.ShapeDtypeStruct((B, D), jnp.float32),
        mesh=mesh,
        scratch_types=scratch_types,
    )(idx_pad, embed_weight)


def _mm_body(w_ref, h_ref, b_ref, out_ref):
    acc = lax.dot_general(w_ref[...], h_ref[...],
                          (((1,), (1,)), ((), ())),
                          preferred_element_type=jnp.float32)
    i = pl.program_id(0)
    vb, nvb = b_ref.shape
    onehot = (lax.broadcasted_iota(jnp.int32, (vb, nvb), 1) == i)
    bcol = jnp.sum(jnp.where(onehot, b_ref[...], 0.0), axis=1, keepdims=True)
    out_ref[...] = acc + bcol


def _project_t(h, w, bias_cols, *, B, D, V, vb, nvb):
    return pl.pallas_call(
        _mm_body,
        grid=(nvb,),
        in_specs=[
            pl.BlockSpec((vb, D), lambda i: (i, 0)),
            pl.BlockSpec((B, D), lambda i: (0, 0)),
            pl.BlockSpec((vb, nvb), lambda i: (0, 0)),
        ],
        out_specs=pl.BlockSpec((vb, B), lambda i: (i, 0)),
        out_shape=jax.ShapeDtypeStruct((V, B), jnp.float32),
        compiler_params=pltpu.CompilerParams(
            dimension_semantics=("arbitrary",)),
    )(w, h, bias_cols)


def kernel(indices, embed_weight, proj_weight, proj_bias):
    B, T = indices.shape
    V, D = embed_weight.shape
    b_per_w = B // _NW
    chunk_b = _CHUNK_B
    chunk = chunk_b * T
    chunk_pad = (chunk + 7) // 8 * 8
    n_chunks = b_per_w // chunk_b

    idx = indices.astype(jnp.int32).reshape(_NW, n_chunks, chunk)
    idx_pad = jnp.pad(idx, ((0, 0), (0, 0), (0, chunk_pad - chunk)))

    h = _pooled_embedding(idx_pad, embed_weight, B=B, T=T, D=D,
                          b_per_w=b_per_w, chunk_b=chunk_b,
                          n_chunks=n_chunks, chunk_pad=chunk_pad)

    vb = _VB
    nvb = pl.cdiv(V, vb)
    bias_cols = jnp.pad(proj_bias, (0, nvb * vb - V)).reshape(nvb, vb).T
    logits_t = _project_t(h, proj_weight, bias_cols, B=B, D=D, V=V,
                          vb=vb, nvb=nvb)
    return logits_t.T
